# Initial kernel scaffold; baseline (speedup 1.0000x reference)
import jax
import jax.numpy as jnp
from jax.experimental import pallas as pl


def kernel(x, table):
    raise NotImplementedError("write your pallas kernel here")


# speedup vs baseline: 2.2899x; 2.2899x over previous
import functools

import jax
import jax.numpy as jnp
from jax import lax
from jax.experimental import pallas as pl
from jax.experimental.pallas import tpu as pltpu
from jax.experimental.pallas import tpu_sc as plsc

VOCAB = 100000
EMBED_DIM = 128
BATCH = 4096
HIST_LEN = 200

NC = 2
NS = 16
NW = NC * NS
NB = BATCH // NW
LPAD = 208
LHALF = LPAD // 2
NL = EMBED_DIM // 16


def _sc_kernel(x_hbm, table_hbm, out_hbm, xbuf, gbuf0, gbuf1, obuf,
               sem0, sem1):
    wid = lax.axis_index("s") * NC + lax.axis_index("c")
    base = wid * NB

    pltpu.sync_copy(x_hbm.at[pl.ds(base, NB)], xbuf)

    def start_gather(j, gbuf, sem):
        c0 = pltpu.make_async_copy(
            table_hbm.at[xbuf.at[j, pl.ds(0, LHALF)]],
            gbuf.at[pl.ds(0, LHALF)], sem)
        c0.start()
        c1 = pltpu.make_async_copy(
            table_hbm.at[xbuf.at[j, pl.ds(LHALF, LHALF)]],
            gbuf.at[pl.ds(LHALF, LHALF)], sem)
        c1.start()
        return c0, c1

    def wait_gather(gbuf, sem):
        c0 = pltpu.make_async_copy(
            table_hbm.at[xbuf.at[0, pl.ds(0, LHALF)]],
            gbuf.at[pl.ds(0, LHALF)], sem)
        c0.wait()
        c1 = pltpu.make_async_copy(
            table_hbm.at[xbuf.at[0, pl.ds(LHALF, LHALF)]],
            gbuf.at[pl.ds(LHALF, LHALF)], sem)
        c1.wait()

    def consume(j, gbuf):
        cnt = jnp.zeros((16,), jnp.float32)
        one = jnp.ones((16,), jnp.float32)
        zero = jnp.zeros((16,), jnp.float32)
        for k in range(LPAD // 16):
            v = xbuf[j, pl.ds(k * 16, 16)]
            cnt = cnt + jnp.where(v != 0, one, zero)
        total = jnp.sum(cnt, axis=0)
        totv = jnp.full((16,), total, jnp.float32)
        inv = one / jnp.maximum(totv, one)

        def body(l, acc):
            return tuple(acc[d] + gbuf[l, pl.ds(d * 16, 16)]
                         for d in range(NL))
        acc = lax.fori_loop(
            0, LPAD, body,
            tuple(jnp.zeros((16,), jnp.float32) for _ in range(NL)))
        for d in range(NL):
            obuf[j, pl.ds(d * 16, 16)] = acc[d] * inv

    bufs = (gbuf0, gbuf1)
    sems = (sem0, sem1)
    start_gather(0, gbuf0, sem0)

    def outer(jj):
        for t in range(2):
            j = jj * 2 + t
            nxt = (t + 1) % 2

            @pl.when(j + 1 < NB)
            def _():
                start_gather(j + 1, bufs[nxt], sems[nxt])

            wait_gather(bufs[t], sems[t])
            consume(j, bufs[t])

    pl.loop(0, NB // 2)(outer)

    pltpu.sync_copy(obuf, out_hbm.at[pl.ds(base, NB)])


@jax.jit
def kernel(x, table):
    xpad = jnp.zeros((BATCH, LPAD), jnp.int32)
    xpad = xpad.at[:, :HIST_LEN].set(x.astype(jnp.int32))
    mesh = plsc.VectorSubcoreMesh(core_axis_name="c", subcore_axis_name="s")
    f = pl.kernel(
        _sc_kernel,
        out_type=jax.ShapeDtypeStruct((BATCH, EMBED_DIM), jnp.float32),
        mesh=mesh,
        compiler_params=pltpu.CompilerParams(
            use_tc_tiling_on_sc=False, needs_layout_passes=False),
        scratch_types=[
            pltpu.VMEM((NB, LPAD), jnp.int32),
            pltpu.VMEM((LPAD, EMBED_DIM), jnp.float32),
            pltpu.VMEM((LPAD, EMBED_DIM), jnp.float32),
            pltpu.VMEM((NB, EMBED_DIM), jnp.float32),
            pltpu.SemaphoreType.DMA,
            pltpu.SemaphoreType.DMA,
        ],
    )
    return f(xpad, table)
